# Initial kernel scaffold; baseline (speedup 1.0000x reference)
import jax
import jax.numpy as jnp
from jax.experimental import pallas as pl


def kernel(x, edge_index, W1l, W1r, b1, W2l, W2r, b2, W3l, W3r, b3):
    raise NotImplementedError("write your pallas kernel here")


# speedup vs baseline: 4.2080x; 4.2080x over previous
import functools

import jax
import jax.numpy as jnp
from jax import lax
from jax.experimental import pallas as pl
from jax.experimental.pallas import tpu as pltpu
from jax.experimental.pallas import tpu_sc as plsc

N_NODES = 10000
N_EDGES = 320000
D_FEAT = 128
HIDDEN = 256
NUM_CLASSES = 64

NC = 2
NS = 16
LANES = 16
K = 80
CH = 80
NCH = N_NODES // CH
CBINS = 640
CB_PS = CBINS // NS
NCK = N_EDGES // (NC * NS) // K
IB = 8
NBLK = (NCK + IB - 1) // IB
NCK_PAD = NBLK * IB
N_ACC = N_NODES + 16


def _make_agg(W: int, with_count: bool):
    mesh = plsc.VectorSubcoreMesh(core_axis_name="c", subcore_axis_name="s")

    out_type = [jax.ShapeDtypeStruct((NC, N_NODES, W), jnp.float32)]
    scratch = [
        pltpu.VMEM((K, W), jnp.float32),
        pltpu.VMEM((K, W), jnp.float32),
        pltpu.VMEM((IB, K), jnp.int32),
        pltpu.VMEM((IB, K), jnp.int32),
        pltpu.SemaphoreType.DMA,
        pltpu.SemaphoreType.DMA,
        pltpu.SemaphoreType.DMA,
        pltpu.SemaphoreType.DMA,
        pltpu.SemaphoreType.DMA,
        pltpu.SemaphoreType.DMA,
        pltpu.VMEM_SHARED((N_ACC, W), jnp.float32),
    ]
    if with_count:
        out_type.append(jax.ShapeDtypeStruct((NC, CBINS, 128), jnp.float32))
        scratch += [
            pltpu.VMEM((K, 128), jnp.float32),
            pltpu.VMEM((K,), jnp.int32),
            pltpu.VMEM_SHARED((CBINS, 128), jnp.float32),
        ]

    @functools.partial(pl.kernel, mesh=mesh, out_type=out_type,
                       scratch_types=scratch)
    def agg(table_hbm, srcm_hbm, dstm_hbm, out_hbm, *rest):
        if with_count:
            cnt_hbm = rest[0]
            (rows_a, rows_b, sidx, didx, is_a, is_b,
             gs_a, gs_b, ss_a, ss_b, acc, crows, divv, cacc) = rest[1:]
        else:
            (rows_a, rows_b, sidx, didx, is_a, is_b,
             gs_a, gs_b, ss_a, ss_b, acc) = rest
        zbuf = rows_a

        cid = lax.axis_index("c")
        sid = lax.axis_index("s")
        wid = sid * NC + cid

        zeros = jnp.zeros((LANES,), jnp.float32)
        iota16 = lax.iota(jnp.int32, LANES)

        @pl.loop(0, K)
        def _(i):
            @pl.loop(0, W, step=LANES)
            def _(j):
                zbuf[i, pl.ds(j, LANES)] = zeros

        if with_count:
            @pl.loop(0, K)
            def _(i):
                @pl.loop(0, 128, step=LANES)
                def _(j):
                    crows[i, pl.ds(j, LANES)] = zeros

            pltpu.sync_copy(crows.at[pl.ds(0, CB_PS)],
                            cacc.at[pl.ds(sid * CB_PS, CB_PS)])

        @pl.loop(0, (NCH + NS - 1) // NS)
        def _(j):
            c = j * NS + sid

            @pl.when(c < NCH)
            def _():
                pltpu.sync_copy(zbuf, acc.at[pl.ds(c * CH, CH)])

        @pl.when(sid == 0)
        def _():
            pltpu.sync_copy(zbuf.at[pl.ds(0, N_ACC - N_NODES)],
                            acc.at[pl.ds(N_NODES, N_ACC - N_NODES)])

        plsc.subcore_barrier()

        def count_rows(jj):
            @pl.loop(0, K, step=LANES)
            def _(i):
                dvec = didx[jj, pl.ds(i, LANES)]
                divv[pl.ds(i, LANES)] = dvec >> 4
                dm = dvec & 15
                for l in range(LANES):
                    crows[i + l, pl.ds(0, LANES)] = jnp.where(
                        iota16 == dm[l], 1.0, 0.0)

            pltpu.sync_copy(crows, cacc.at[divv], add=True)

        @pl.loop(0, NBLK)
        def _(b):
            h0 = pltpu.async_copy(srcm_hbm.at[wid, pl.ds(b * IB, IB)],
                                  sidx, is_a)
            h1 = pltpu.async_copy(dstm_hbm.at[wid, pl.ds(b * IB, IB)],
                                  didx, is_b)
            h0.wait()
            h1.wait()
            sa = sb = None
            for j in range(IB // 2):
                if sa is not None:
                    sa.wait()
                ga = pltpu.async_copy(table_hbm.at[sidx.at[2 * j]],
                                      rows_a, gs_a)
                if sb is not None:
                    sb.wait()
                gb = pltpu.async_copy(table_hbm.at[sidx.at[2 * j + 1]],
                                      rows_b, gs_b)
                if with_count:
                    count_rows(2 * j)
                ga.wait()
                sa = pltpu.async_copy(rows_a, acc.at[didx.at[2 * j]],
                                      ss_a, add=True)
                if with_count:
                    count_rows(2 * j + 1)
                gb.wait()
                sb = pltpu.async_copy(rows_b, acc.at[didx.at[2 * j + 1]],
                                      ss_b, add=True)
            sa.wait()
            sb.wait()

        plsc.subcore_barrier()

        @pl.loop(0, (NCH + NS - 1) // NS)
        def _(j):
            c = j * NS + sid

            @pl.when(c < NCH)
            def _():
                pltpu.sync_copy(acc.at[pl.ds(c * CH, CH)],
                                out_hbm.at[cid, pl.ds(c * CH, CH)])

        if with_count:
            pltpu.sync_copy(cacc.at[pl.ds(sid * CB_PS, CB_PS)],
                            cnt_hbm.at[cid, pl.ds(sid * CB_PS, CB_PS)])

    if with_count:
        return agg
    return lambda *a: agg(*a)[0]


NB = 1000


def _invc(cntp_ref):
    cnt = cntp_ref[0] + cntp_ref[1]
    return 1.0 / jnp.maximum(cnt, 1.0)


def _lin1_body(aggp, cntp, x, w1l, w1r, b1, out):
    mean = (aggp[0] + aggp[1]) * _invc(cntp)
    h = (jnp.dot(mean, w1l[...], preferred_element_type=jnp.float32)
         + jnp.dot(x[...], w1r[...], preferred_element_type=jnp.float32)
         + b1[...])
    h = jnp.maximum(h, 0.0)
    out[0] = h[:, :D_FEAT]
    out[1] = h[:, D_FEAT:]


def _lin2_body(a2a, a2b, cntp, h1, w2l, w2r, b2, w3l, w3r, b3, p_out, s_out):
    invc = _invc(cntp)
    mean = jnp.concatenate([(a2a[0] + a2a[1]) * invc,
                            (a2b[0] + a2b[1]) * invc], axis=1)
    hin = jnp.concatenate([h1[0], h1[1]], axis=1)
    h = (jnp.dot(mean, w2l[...], preferred_element_type=jnp.float32)
         + jnp.dot(hin, w2r[...], preferred_element_type=jnp.float32)
         + b2[...])
    h = jnp.maximum(h, 0.0)
    p_out[...] = jnp.dot(h, w3l[...], preferred_element_type=jnp.float32)
    s_out[...] = jnp.dot(h, w3r[...], preferred_element_type=jnp.float32) + b3[...]


def _fin_body(a3, cntp, s, out):
    out[...] = (a3[0, :, :NUM_CLASSES] + a3[1, :, :NUM_CLASSES]) * _invc(cntp) + s[...]


def _full(shape):
    return pl.BlockSpec(shape, lambda i: (0,) * len(shape))


def _rows(shape):
    if len(shape) == 3:
        return pl.BlockSpec(shape, lambda i: (0, i, 0))
    return pl.BlockSpec(shape, lambda i: (i, 0))


def kernel(x, edge_index, W1l, W1r, b1, W2l, W2r, b2, W3l, W3r, b3):
    ei = edge_index.astype(jnp.int32)
    src, dst = ei[0], ei[1]
    srcm = jnp.pad(src.reshape(NC * NS, NCK, K),
                   ((0, 0), (0, NCK_PAD - NCK), (0, 0)))
    dstm = jnp.pad(dst.reshape(NC * NS, NCK, K),
                   ((0, 0), (0, NCK_PAD - NCK), (0, 0)),
                   constant_values=N_NODES)
    b1r, b2r, b3r = b1.reshape(1, -1), b2.reshape(1, -1), b3.reshape(1, -1)
    W3lp = jnp.pad(W3l, ((0, 0), (0, D_FEAT - NUM_CLASSES)))

    agg1p, cbins = _make_agg(D_FEAT, True)(x, srcm, dstm)
    cntp = cbins[:, :N_NODES // 16, :16].reshape(2, N_NODES, 1)

    h1fm = pl.pallas_call(
        _lin1_body,
        grid=(N_NODES // NB,),
        in_specs=[_rows((2, NB, D_FEAT)), _rows((2, NB, 1)),
                  _rows((NB, D_FEAT)), _full((D_FEAT, HIDDEN)),
                  _full((D_FEAT, HIDDEN)), _full((1, HIDDEN))],
        out_specs=_rows((2, NB, D_FEAT)),
        out_shape=jax.ShapeDtypeStruct((2, N_NODES, D_FEAT), jnp.float32),
    )(agg1p, cntp, x, W1l, W1r, b1r)

    agg2 = _make_agg(D_FEAT, False)
    a2a = agg2(h1fm[0], srcm, dstm)
    a2b = agg2(h1fm[1], srcm, dstm)

    p, s = pl.pallas_call(
        _lin2_body,
        grid=(N_NODES // NB,),
        in_specs=[_rows((2, NB, D_FEAT)), _rows((2, NB, D_FEAT)),
                  _rows((2, NB, 1)), _rows((2, NB, D_FEAT)),
                  _full((HIDDEN, 2 * HIDDEN)), _full((HIDDEN, 2 * HIDDEN)),
                  _full((1, 2 * HIDDEN)), _full((2 * HIDDEN, D_FEAT)),
                  _full((2 * HIDDEN, NUM_CLASSES)), _full((1, NUM_CLASSES))],
        out_specs=[_rows((NB, D_FEAT)), _rows((NB, NUM_CLASSES))],
        out_shape=[jax.ShapeDtypeStruct((N_NODES, D_FEAT), jnp.float32),
                   jax.ShapeDtypeStruct((N_NODES, NUM_CLASSES), jnp.float32)],
    )(a2a, a2b, cntp, h1fm, W2l, W2r, b2r, W3lp, W3r, b3r)

    a3 = _make_agg(D_FEAT, False)(p, srcm, dstm)

    out = pl.pallas_call(
        _fin_body,
        grid=(N_NODES // NB,),
        in_specs=[_rows((2, NB, D_FEAT)), _rows((2, NB, 1)),
                  _rows((NB, NUM_CLASSES))],
        out_specs=_rows((NB, NUM_CLASSES)),
        out_shape=jax.ShapeDtypeStruct((N_NODES, NUM_CLASSES), jnp.float32),
    )(a3, cntp, s)

    return out


# speedup vs baseline: 9.3185x; 2.2145x over previous
import functools

import jax
import jax.numpy as jnp
from jax import lax
from jax.experimental import pallas as pl
from jax.experimental.pallas import tpu as pltpu
from jax.experimental.pallas import tpu_sc as plsc

N_NODES = 10000
N_EDGES = 320000
D_FEAT = 128
HIDDEN = 256
NUM_CLASSES = 64

NC = 2
NS = 16
LANES = 16
K = 80
CH = 80
NCH = N_NODES // CH
CBINS = 640
CB_PS = CBINS // NS


def _make_agg(W: int, with_count: bool):
    epw = N_EDGES // (NC * NS)
    nchunks = epw // K
    npairs = nchunks // 2
    mesh = plsc.VectorSubcoreMesh(core_axis_name="c", subcore_axis_name="s")

    out_type = [jax.ShapeDtypeStruct((NC, N_NODES, W), jnp.float32)]
    scratch = [
        pltpu.VMEM((K, W), jnp.float32),
        pltpu.VMEM((K, W), jnp.float32),
        pltpu.VMEM((K,), jnp.int32),
        pltpu.VMEM((K,), jnp.int32),
        pltpu.VMEM((K,), jnp.int32),
        pltpu.VMEM((K,), jnp.int32),
        pltpu.SemaphoreType.DMA,
        pltpu.SemaphoreType.DMA,
        pltpu.SemaphoreType.DMA,
        pltpu.SemaphoreType.DMA,
        pltpu.SemaphoreType.DMA,
        pltpu.SemaphoreType.DMA,
        pltpu.SemaphoreType.DMA,
        pltpu.SemaphoreType.DMA,
        pltpu.VMEM_SHARED((N_NODES, W), jnp.float32),
    ]
    if with_count:
        out_type.append(jax.ShapeDtypeStruct((NC, CBINS, 128), jnp.float32))
        scratch += [
            pltpu.VMEM((K, 128), jnp.float32),
            pltpu.VMEM((K,), jnp.int32),
            pltpu.VMEM_SHARED((CBINS, 128), jnp.float32),
        ]

    @functools.partial(pl.kernel, mesh=mesh, out_type=out_type,
                       scratch_types=scratch)
    def agg(table_hbm, src_hbm, dst_hbm, out_hbm, *rest):
        if with_count:
            cnt_hbm = rest[0]
            (rows_a, rows_b, src_a, dst_a, src_b, dst_b,
             gs_a, gs_b, ss_a, ss_b, is_a, id_a, is_b, id_b,
             acc, crows, divv, cacc) = rest[1:]
        else:
            (rows_a, rows_b, src_a, dst_a, src_b, dst_b,
             gs_a, gs_b, ss_a, ss_b, is_a, id_a, is_b, id_b, acc) = rest
        zbuf = rows_a

        cid = lax.axis_index("c")
        sid = lax.axis_index("s")
        wid = sid * NC + cid
        base = wid * epw

        zeros = jnp.zeros((LANES,), jnp.float32)
        iota16 = lax.iota(jnp.int32, LANES)

        @pl.loop(0, K)
        def _(i):
            @pl.loop(0, W, step=LANES)
            def _(j):
                zbuf[i, pl.ds(j, LANES)] = zeros

        if with_count:
            @pl.loop(0, K)
            def _(i):
                @pl.loop(0, 128, step=LANES)
                def _(j):
                    crows[i, pl.ds(j, LANES)] = zeros

            pltpu.sync_copy(crows.at[pl.ds(0, CB_PS)],
                            cacc.at[pl.ds(sid * CB_PS, CB_PS)])

        @pl.loop(0, (NCH + NS - 1) // NS)
        def _(j):
            c = j * NS + sid

            @pl.when(c < NCH)
            def _():
                pltpu.sync_copy(zbuf, acc.at[pl.ds(c * CH, CH)])

        plsc.subcore_barrier()

        def load_idx(c, sv, dv):
            pltpu.sync_copy(src_hbm.at[pl.ds(base + c * K, K)], sv)
            pltpu.sync_copy(dst_hbm.at[pl.ds(base + c * K, K)], dv)

        def count_rows(dv):
            @pl.loop(0, K, step=LANES)
            def _(i):
                dvec = dv[pl.ds(i, LANES)]
                divv[pl.ds(i, LANES)] = dvec >> 4
                dm = dvec & 15
                for l in range(LANES):
                    crows[i + l, pl.ds(0, LANES)] = jnp.where(
                        iota16 == dm[l], 1.0, 0.0)

            pltpu.sync_copy(crows, cacc.at[divv], add=True)

        @pl.loop(0, npairs)
        def _(p):
            c0 = p * 2
            i0 = pltpu.async_copy(src_hbm.at[pl.ds(base + c0 * K, K)],
                                  src_a, is_a)
            i1 = pltpu.async_copy(dst_hbm.at[pl.ds(base + c0 * K, K)],
                                  dst_a, id_a)
            i2 = pltpu.async_copy(src_hbm.at[pl.ds(base + (c0 + 1) * K, K)],
                                  src_b, is_b)
            i3 = pltpu.async_copy(dst_hbm.at[pl.ds(base + (c0 + 1) * K, K)],
                                  dst_b, id_b)
            i0.wait()
            g0 = pltpu.async_copy(table_hbm.at[src_a], rows_a, gs_a)
            i2.wait()
            g1 = pltpu.async_copy(table_hbm.at[src_b], rows_b, gs_b)
            i1.wait()
            i3.wait()
            if with_count:
                count_rows(dst_a)
            g0.wait()
            s0 = pltpu.async_copy(rows_a, acc.at[dst_a], ss_a, add=True)
            if with_count:
                count_rows(dst_b)
            g1.wait()
            s1 = pltpu.async_copy(rows_b, acc.at[dst_b], ss_b, add=True)
            s0.wait()
            s1.wait()

        if nchunks % 2:
            c = nchunks - 1
            load_idx(c, src_a, dst_a)
            pltpu.sync_copy(table_hbm.at[src_a], rows_a)
            pltpu.sync_copy(rows_a, acc.at[dst_a], add=True)
            if with_count:
                count_rows(dst_a)

        plsc.subcore_barrier()

        @pl.loop(0, (NCH + NS - 1) // NS)
        def _(j):
            c = j * NS + sid

            @pl.when(c < NCH)
            def _():
                pltpu.sync_copy(acc.at[pl.ds(c * CH, CH)],
                                out_hbm.at[cid, pl.ds(c * CH, CH)])

        if with_count:
            pltpu.sync_copy(cacc.at[pl.ds(sid * CB_PS, CB_PS)],
                            cnt_hbm.at[cid, pl.ds(sid * CB_PS, CB_PS)])

    if with_count:
        return agg
    return lambda *a: agg(*a)[0]


NB = 1000


def _invc(cntp_ref):
    cnt = cntp_ref[0] + cntp_ref[1]
    return 1.0 / jnp.maximum(cnt, 1.0)


def _lin1_body(aggp, cntp, x, w1l, w1r, b1, out):
    mean = (aggp[0] + aggp[1]) * _invc(cntp)
    h = (jnp.dot(mean, w1l[...], preferred_element_type=jnp.float32)
         + jnp.dot(x[...], w1r[...], preferred_element_type=jnp.float32)
         + b1[...])
    h = jnp.maximum(h, 0.0)
    out[0] = h[:, :D_FEAT]
    out[1] = h[:, D_FEAT:]


def _lin2_body(a2a, a2b, cntp, h1, w2l, w2r, b2, w3l, w3r, b3, p_out, s_out):
    invc = _invc(cntp)
    mean = jnp.concatenate([(a2a[0] + a2a[1]) * invc,
                            (a2b[0] + a2b[1]) * invc], axis=1)
    hin = jnp.concatenate([h1[0], h1[1]], axis=1)
    h = (jnp.dot(mean, w2l[...], preferred_element_type=jnp.float32)
         + jnp.dot(hin, w2r[...], preferred_element_type=jnp.float32)
         + b2[...])
    h = jnp.maximum(h, 0.0)
    p_out[...] = jnp.dot(h, w3l[...], preferred_element_type=jnp.float32)
    s_out[...] = jnp.dot(h, w3r[...], preferred_element_type=jnp.float32) + b3[...]


def _fin_body(a3, cntp, s, out):
    out[...] = (a3[0, :, :NUM_CLASSES] + a3[1, :, :NUM_CLASSES]) * _invc(cntp) + s[...]


def _full(shape):
    return pl.BlockSpec(shape, lambda i: (0,) * len(shape))


def _rows(shape):
    if len(shape) == 3:
        return pl.BlockSpec(shape, lambda i: (0, i, 0))
    return pl.BlockSpec(shape, lambda i: (i, 0))


def kernel(x, edge_index, W1l, W1r, b1, W2l, W2r, b2, W3l, W3r, b3):
    ei = edge_index.astype(jnp.int32)
    src, dst = ei[0], ei[1]
    b1r, b2r, b3r = b1.reshape(1, -1), b2.reshape(1, -1), b3.reshape(1, -1)
    W3lp = jnp.pad(W3l, ((0, 0), (0, D_FEAT - NUM_CLASSES)))

    agg1p, cbins = _make_agg(D_FEAT, True)(x, src, dst)
    cntp = cbins[:, :N_NODES // 16, :16].reshape(2, N_NODES, 1)

    h1fm = pl.pallas_call(
        _lin1_body,
        grid=(N_NODES // NB,),
        in_specs=[_rows((2, NB, D_FEAT)), _rows((2, NB, 1)),
                  _rows((NB, D_FEAT)), _full((D_FEAT, HIDDEN)),
                  _full((D_FEAT, HIDDEN)), _full((1, HIDDEN))],
        out_specs=_rows((2, NB, D_FEAT)),
        out_shape=jax.ShapeDtypeStruct((2, N_NODES, D_FEAT), jnp.float32),
    )(agg1p, cntp, x, W1l, W1r, b1r)

    agg2 = _make_agg(D_FEAT, False)
    a2a = agg2(h1fm[0], src, dst)
    a2b = agg2(h1fm[1], src, dst)

    p, s = pl.pallas_call(
        _lin2_body,
        grid=(N_NODES // NB,),
        in_specs=[_rows((2, NB, D_FEAT)), _rows((2, NB, D_FEAT)),
                  _rows((2, NB, 1)), _rows((2, NB, D_FEAT)),
                  _full((HIDDEN, 2 * HIDDEN)), _full((HIDDEN, 2 * HIDDEN)),
                  _full((1, 2 * HIDDEN)), _full((2 * HIDDEN, D_FEAT)),
                  _full((2 * HIDDEN, NUM_CLASSES)), _full((1, NUM_CLASSES))],
        out_specs=[_rows((NB, D_FEAT)), _rows((NB, NUM_CLASSES))],
        out_shape=[jax.ShapeDtypeStruct((N_NODES, D_FEAT), jnp.float32),
                   jax.ShapeDtypeStruct((N_NODES, NUM_CLASSES), jnp.float32)],
    )(a2a, a2b, cntp, h1fm, W2l, W2r, b2r, W3lp, W3r, b3r)

    a3 = _make_agg(D_FEAT, False)(p, src, dst)

    out = pl.pallas_call(
        _fin_body,
        grid=(N_NODES // NB,),
        in_specs=[_rows((2, NB, D_FEAT)), _rows((2, NB, 1)),
                  _rows((NB, NUM_CLASSES))],
        out_specs=_rows((NB, NUM_CLASSES)),
        out_shape=jax.ShapeDtypeStruct((N_NODES, NUM_CLASSES), jnp.float32),
    )(a3, cntp, s)

    return out


# speedup vs baseline: 11.1454x; 1.1961x over previous
import functools

import jax
import jax.numpy as jnp
from jax import lax
from jax.experimental import pallas as pl
from jax.experimental.pallas import tpu as pltpu
from jax.experimental.pallas import tpu_sc as plsc

N_NODES = 10000
N_EDGES = 320000
D_FEAT = 128
HIDDEN = 256
NUM_CLASSES = 64

NC = 2
NS = 16
LANES = 16
K = 80
CH = 80
NCH = N_NODES // CH
CBINS = 640
CB_PS = CBINS // NS


def _make_agg(W: int, with_count: bool):
    epw = N_EDGES // (NC * NS)
    nchunks = epw // K
    npairs = nchunks // 2
    mesh = plsc.VectorSubcoreMesh(core_axis_name="c", subcore_axis_name="s")

    out_type = [jax.ShapeDtypeStruct((NC, N_NODES, W), jnp.float32)]
    scratch = [
        pltpu.VMEM((K, W), jnp.float32),
        pltpu.VMEM((K, W), jnp.float32),
        pltpu.VMEM((K,), jnp.int32),
        pltpu.VMEM((K,), jnp.int32),
        pltpu.VMEM((K,), jnp.int32),
        pltpu.VMEM((K,), jnp.int32),
        pltpu.VMEM((K,), jnp.int32),
        pltpu.VMEM((K,), jnp.int32),
        pltpu.VMEM((K,), jnp.int32),
        pltpu.VMEM((K,), jnp.int32),
        pltpu.SemaphoreType.DMA,
        pltpu.SemaphoreType.DMA,
        pltpu.SemaphoreType.DMA,
        pltpu.SemaphoreType.DMA,
        pltpu.SemaphoreType.DMA,
        pltpu.SemaphoreType.DMA,
        pltpu.SemaphoreType.DMA,
        pltpu.SemaphoreType.DMA,
        pltpu.VMEM_SHARED((N_NODES, W), jnp.float32),
    ]
    if with_count:
        out_type.append(jax.ShapeDtypeStruct((NC, CBINS, 128), jnp.float32))
        scratch += [
            pltpu.VMEM((K, 128), jnp.float32),
            pltpu.VMEM((K,), jnp.int32),
            pltpu.VMEM_SHARED((CBINS, 128), jnp.float32),
        ]

    @functools.partial(pl.kernel, mesh=mesh, out_type=out_type,
                       scratch_types=scratch)
    def agg(table_hbm, src_hbm, dst_hbm, out_hbm, *rest):
        if with_count:
            cnt_hbm = rest[0]
            (rows_a, rows_b, sax, dax, sbx, dbx, say, day, sby, dby,
             gs_a, gs_b, ss_a, ss_b, is_a, id_a, is_b, id_b,
             acc, crows, divv, cacc) = rest[1:]
        else:
            (rows_a, rows_b, sax, dax, sbx, dbx, say, day, sby, dby,
             gs_a, gs_b, ss_a, ss_b, is_a, id_a, is_b, id_b, acc) = rest
        set_x = (sax, dax, sbx, dbx)
        set_y = (say, day, sby, dby)
        zbuf = rows_a

        cid = lax.axis_index("c")
        sid = lax.axis_index("s")
        wid = sid * NC + cid
        base = wid * epw

        zeros = jnp.zeros((LANES,), jnp.float32)
        iota16 = lax.iota(jnp.int32, LANES)

        @pl.loop(0, K)
        def _(i):
            @pl.loop(0, W, step=LANES)
            def _(j):
                zbuf[i, pl.ds(j, LANES)] = zeros

        if with_count:
            @pl.loop(0, K)
            def _(i):
                @pl.loop(0, 128, step=LANES)
                def _(j):
                    crows[i, pl.ds(j, LANES)] = zeros

            pltpu.sync_copy(crows.at[pl.ds(0, CB_PS)],
                            cacc.at[pl.ds(sid * CB_PS, CB_PS)])

        @pl.loop(0, (NCH + NS - 1) // NS)
        def _(j):
            c = j * NS + sid

            @pl.when(c < NCH)
            def _():
                pltpu.sync_copy(zbuf, acc.at[pl.ds(c * CH, CH)])

        plsc.subcore_barrier()

        isems = (is_a, id_a, is_b, id_b)

        def prefetch(p, bufs):
            off = base + p * 2 * K
            pltpu.async_copy(src_hbm.at[pl.ds(off, K)], bufs[0], is_a)
            pltpu.async_copy(dst_hbm.at[pl.ds(off, K)], bufs[1], id_a)
            pltpu.async_copy(src_hbm.at[pl.ds(off + K, K)], bufs[2], is_b)
            pltpu.async_copy(dst_hbm.at[pl.ds(off + K, K)], bufs[3], id_b)

        def wait_idx(bufs):
            for buf, sem in zip(bufs, isems):
                pltpu.make_async_copy(src_hbm.at[pl.ds(base, K)], buf,
                                      sem).wait()

        def drain_scatter(sem, rows):
            pltpu.make_async_copy(table_hbm.at[pl.ds(0, K)], rows,
                                  sem).wait()

        def count_rows(dv):
            @pl.loop(0, K, step=LANES)
            def _(i):
                dvec = dv[pl.ds(i, LANES)]
                divv[pl.ds(i, LANES)] = dvec >> 4
                dm = dvec & 15
                for l in range(LANES):
                    crows[i + l, pl.ds(0, LANES)] = jnp.where(
                        iota16 == dm[l], 1.0, 0.0)

            pltpu.sync_copy(crows, cacc.at[divv], add=True)

        prefetch(0, set_x)

        @pl.loop(0, npairs // 2)
        def _(q):
            wait_idx(set_x)

            @pl.when(q > 0)
            def _():
                drain_scatter(ss_a, rows_a)

            g0 = pltpu.async_copy(table_hbm.at[sax], rows_a, gs_a)

            @pl.when(q > 0)
            def _():
                drain_scatter(ss_b, rows_b)

            g1 = pltpu.async_copy(table_hbm.at[sbx], rows_b, gs_b)
            prefetch(2 * q + 1, set_y)
            if with_count:
                count_rows(dax)
            g0.wait()
            pltpu.async_copy(rows_a, acc.at[dax], ss_a, add=True)
            if with_count:
                count_rows(dbx)
            g1.wait()
            pltpu.async_copy(rows_b, acc.at[dbx], ss_b, add=True)

            wait_idx(set_y)
            drain_scatter(ss_a, rows_a)
            g0 = pltpu.async_copy(table_hbm.at[say], rows_a, gs_a)
            drain_scatter(ss_b, rows_b)
            g1 = pltpu.async_copy(table_hbm.at[sby], rows_b, gs_b)
            prefetch(2 * q + 2, set_x)
            if with_count:
                count_rows(day)
            g0.wait()
            pltpu.async_copy(rows_a, acc.at[day], ss_a, add=True)
            if with_count:
                count_rows(dby)
            g1.wait()
            pltpu.async_copy(rows_b, acc.at[dby], ss_b, add=True)

        wait_idx(set_x)
        drain_scatter(ss_a, rows_a)
        drain_scatter(ss_b, rows_b)
        pltpu.sync_copy(table_hbm.at[sax], rows_a)
        pltpu.sync_copy(rows_a, acc.at[dax], add=True)
        if with_count:
            count_rows(dax)

        plsc.subcore_barrier()

        @pl.loop(0, (NCH + NS - 1) // NS)
        def _(j):
            c = j * NS + sid

            @pl.when(c < NCH)
            def _():
                pltpu.sync_copy(acc.at[pl.ds(c * CH, CH)],
                                out_hbm.at[cid, pl.ds(c * CH, CH)])

        if with_count:
            pltpu.sync_copy(cacc.at[pl.ds(sid * CB_PS, CB_PS)],
                            cnt_hbm.at[cid, pl.ds(sid * CB_PS, CB_PS)])

    if with_count:
        return agg
    return lambda *a: agg(*a)[0]


NB = 1000


def _invc(cntp_ref):
    cnt = cntp_ref[0] + cntp_ref[1]
    return 1.0 / jnp.maximum(cnt, 1.0)


def _lin1_body(aggp, cntp, x, w1l, w1r, b1, out):
    mean = (aggp[0] + aggp[1]) * _invc(cntp)
    h = (jnp.dot(mean, w1l[...], preferred_element_type=jnp.float32)
         + jnp.dot(x[...], w1r[...], preferred_element_type=jnp.float32)
         + b1[...])
    h = jnp.maximum(h, 0.0)
    out[0] = h[:, :D_FEAT]
    out[1] = h[:, D_FEAT:]


def _lin2_body(a2a, a2b, cntp, h1, w2l, w2r, b2, w3l, w3r, b3, p_out, s_out):
    invc = _invc(cntp)
    mean = jnp.concatenate([(a2a[0] + a2a[1]) * invc,
                            (a2b[0] + a2b[1]) * invc], axis=1)
    hin = jnp.concatenate([h1[0], h1[1]], axis=1)
    h = (jnp.dot(mean, w2l[...], preferred_element_type=jnp.float32)
         + jnp.dot(hin, w2r[...], preferred_element_type=jnp.float32)
         + b2[...])
    h = jnp.maximum(h, 0.0)
    p_out[...] = jnp.dot(h, w3l[...], preferred_element_type=jnp.float32)
    s_out[...] = jnp.dot(h, w3r[...], preferred_element_type=jnp.float32) + b3[...]


def _fin_body(a3, cntp, s, out):
    out[...] = (a3[0, :, :NUM_CLASSES] + a3[1, :, :NUM_CLASSES]) * _invc(cntp) + s[...]


def _full(shape):
    return pl.BlockSpec(shape, lambda i: (0,) * len(shape))


def _rows(shape):
    if len(shape) == 3:
        return pl.BlockSpec(shape, lambda i: (0, i, 0))
    return pl.BlockSpec(shape, lambda i: (i, 0))


def kernel(x, edge_index, W1l, W1r, b1, W2l, W2r, b2, W3l, W3r, b3):
    ei = edge_index.astype(jnp.int32)
    src = jnp.pad(ei[0], (0, 2 * K))
    dst = jnp.pad(ei[1], (0, 2 * K))
    b1r, b2r, b3r = b1.reshape(1, -1), b2.reshape(1, -1), b3.reshape(1, -1)
    W3lp = jnp.pad(W3l, ((0, 0), (0, D_FEAT - NUM_CLASSES)))

    agg1p, cbins = _make_agg(D_FEAT, True)(x, src, dst)
    cntp = cbins[:, :N_NODES // 16, :16].reshape(2, N_NODES, 1)

    h1fm = pl.pallas_call(
        _lin1_body,
        grid=(N_NODES // NB,),
        in_specs=[_rows((2, NB, D_FEAT)), _rows((2, NB, 1)),
                  _rows((NB, D_FEAT)), _full((D_FEAT, HIDDEN)),
                  _full((D_FEAT, HIDDEN)), _full((1, HIDDEN))],
        out_specs=_rows((2, NB, D_FEAT)),
        out_shape=jax.ShapeDtypeStruct((2, N_NODES, D_FEAT), jnp.float32),
    )(agg1p, cntp, x, W1l, W1r, b1r)

    agg2 = _make_agg(D_FEAT, False)
    a2a = agg2(h1fm[0], src, dst)
    a2b = agg2(h1fm[1], src, dst)

    p, s = pl.pallas_call(
        _lin2_body,
        grid=(N_NODES // NB,),
        in_specs=[_rows((2, NB, D_FEAT)), _rows((2, NB, D_FEAT)),
                  _rows((2, NB, 1)), _rows((2, NB, D_FEAT)),
                  _full((HIDDEN, 2 * HIDDEN)), _full((HIDDEN, 2 * HIDDEN)),
                  _full((1, 2 * HIDDEN)), _full((2 * HIDDEN, D_FEAT)),
                  _full((2 * HIDDEN, NUM_CLASSES)), _full((1, NUM_CLASSES))],
        out_specs=[_rows((NB, D_FEAT)), _rows((NB, NUM_CLASSES))],
        out_shape=[jax.ShapeDtypeStruct((N_NODES, D_FEAT), jnp.float32),
                   jax.ShapeDtypeStruct((N_NODES, NUM_CLASSES), jnp.float32)],
    )(a2a, a2b, cntp, h1fm, W2l, W2r, b2r, W3lp, W3r, b3r)

    a3 = _make_agg(D_FEAT, False)(p, src, dst)

    out = pl.pallas_call(
        _fin_body,
        grid=(N_NODES // NB,),
        in_specs=[_rows((2, NB, D_FEAT)), _rows((2, NB, 1)),
                  _rows((NB, NUM_CLASSES))],
        out_specs=_rows((NB, NUM_CLASSES)),
        out_shape=jax.ShapeDtypeStruct((N_NODES, NUM_CLASSES), jnp.float32),
    )(a3, cntp, s)

    return out


# speedup vs baseline: 11.1542x; 1.0008x over previous
import functools

import jax
import jax.numpy as jnp
from jax import lax
from jax.experimental import pallas as pl
from jax.experimental.pallas import tpu as pltpu
from jax.experimental.pallas import tpu_sc as plsc

N_NODES = 10000
N_EDGES = 320000
D_FEAT = 128
HIDDEN = 256
NUM_CLASSES = 64

NC = 2
NS = 16
LANES = 16
K = 80
CH = 80
NCH = N_NODES // CH
CBINS = 640
CB_PS = CBINS // NS


def _make_agg(W: int, with_count: bool):
    epw = N_EDGES // (NC * NS)
    nchunks = epw // K
    npairs = nchunks // 2
    mesh = plsc.VectorSubcoreMesh(core_axis_name="c", subcore_axis_name="s")

    out_type = [jax.ShapeDtypeStruct((NC, N_NODES, W), jnp.float32)]
    scratch = [
        pltpu.VMEM((K, W), jnp.float32),
        pltpu.VMEM((K, W), jnp.float32),
        pltpu.VMEM((K,), jnp.int32),
        pltpu.VMEM((K,), jnp.int32),
        pltpu.VMEM((K,), jnp.int32),
        pltpu.VMEM((K,), jnp.int32),
        pltpu.VMEM((K,), jnp.int32),
        pltpu.VMEM((K,), jnp.int32),
        pltpu.VMEM((K,), jnp.int32),
        pltpu.VMEM((K,), jnp.int32),
        pltpu.SemaphoreType.DMA,
        pltpu.SemaphoreType.DMA,
        pltpu.SemaphoreType.DMA,
        pltpu.SemaphoreType.DMA,
        pltpu.SemaphoreType.DMA,
        pltpu.SemaphoreType.DMA,
        pltpu.SemaphoreType.DMA,
        pltpu.SemaphoreType.DMA,
        pltpu.VMEM_SHARED((N_NODES, W), jnp.float32),
    ]
    if with_count:
        out_type.append(jax.ShapeDtypeStruct((NC, CBINS, 128), jnp.float32))
        scratch += [
            pltpu.VMEM((K, 128), jnp.float32),
            pltpu.VMEM((K,), jnp.int32),
            pltpu.VMEM_SHARED((CBINS, 128), jnp.float32),
        ]

    @functools.partial(pl.kernel, mesh=mesh, out_type=out_type,
                       scratch_types=scratch)
    def agg(table_hbm, src_hbm, dst_hbm, out_hbm, *rest):
        if with_count:
            cnt_hbm = rest[0]
            (rows_a, rows_b, sax, dax, sbx, dbx, say, day, sby, dby,
             gs_a, gs_b, ss_a, ss_b, is_a, id_a, is_b, id_b,
             acc, crows, divv, cacc) = rest[1:]
        else:
            (rows_a, rows_b, sax, dax, sbx, dbx, say, day, sby, dby,
             gs_a, gs_b, ss_a, ss_b, is_a, id_a, is_b, id_b, acc) = rest
        set_x = (sax, dax, sbx, dbx)
        set_y = (say, day, sby, dby)
        zbuf = rows_a

        cid = lax.axis_index("c")
        sid = lax.axis_index("s")
        wid = sid * NC + cid
        base = wid * epw

        zeros = jnp.zeros((LANES,), jnp.float32)
        iota16 = lax.iota(jnp.int32, LANES)

        @pl.loop(0, K)
        def _(i):
            @pl.loop(0, W, step=LANES)
            def _(j):
                zbuf[i, pl.ds(j, LANES)] = zeros

        if with_count:
            @pl.loop(0, K)
            def _(i):
                @pl.loop(0, 128, step=LANES)
                def _(j):
                    crows[i, pl.ds(j, LANES)] = zeros

            pltpu.sync_copy(crows.at[pl.ds(0, CB_PS)],
                            cacc.at[pl.ds(sid * CB_PS, CB_PS)])

        @pl.loop(0, (NCH + NS - 1) // NS)
        def _(j):
            c = j * NS + sid

            @pl.when(c < NCH)
            def _():
                pltpu.sync_copy(zbuf, acc.at[pl.ds(c * CH, CH)])

        plsc.subcore_barrier()

        isems = (is_a, id_a, is_b, id_b)

        def prefetch(p, bufs):
            off = base + p * 2 * K
            pltpu.async_copy(src_hbm.at[pl.ds(off, K)], bufs[0], is_a)
            pltpu.async_copy(dst_hbm.at[pl.ds(off, K)], bufs[1], id_a)
            pltpu.async_copy(src_hbm.at[pl.ds(off + K, K)], bufs[2], is_b)
            pltpu.async_copy(dst_hbm.at[pl.ds(off + K, K)], bufs[3], id_b)

        def wait_idx(bufs):
            for buf, sem in zip(bufs, isems):
                pltpu.make_async_copy(src_hbm.at[pl.ds(base, K)], buf,
                                      sem).wait()

        def drain_scatter(sem, rows):
            pltpu.make_async_copy(table_hbm.at[pl.ds(0, K)], rows,
                                  sem).wait()

        def count_rows(dv):
            @pl.loop(0, K, step=LANES)
            def _(i):
                dvec = dv[pl.ds(i, LANES)]
                divv[pl.ds(i, LANES)] = dvec >> 4
                dm = dvec & 15
                for l in range(LANES):
                    crows[i + l, pl.ds(0, LANES)] = jnp.where(
                        iota16 == dm[l], 1.0, 0.0)

            pltpu.sync_copy(crows, cacc.at[divv], add=True)

        prefetch(0, set_x)

        @pl.loop(0, npairs // 2)
        def _(q):
            wait_idx(set_x)

            @pl.when(q > 0)
            def _():
                drain_scatter(ss_a, rows_a)

            g0 = pltpu.async_copy(table_hbm.at[sax], rows_a, gs_a)

            @pl.when(q > 0)
            def _():
                drain_scatter(ss_b, rows_b)

            g1 = pltpu.async_copy(table_hbm.at[sbx], rows_b, gs_b)
            prefetch(2 * q + 1, set_y)
            if with_count:
                count_rows(dax)
            g0.wait()
            pltpu.async_copy(rows_a, acc.at[dax], ss_a, add=True)
            if with_count:
                count_rows(dbx)
            g1.wait()
            pltpu.async_copy(rows_b, acc.at[dbx], ss_b, add=True)

            wait_idx(set_y)
            drain_scatter(ss_a, rows_a)
            g0 = pltpu.async_copy(table_hbm.at[say], rows_a, gs_a)
            drain_scatter(ss_b, rows_b)
            g1 = pltpu.async_copy(table_hbm.at[sby], rows_b, gs_b)
            prefetch(2 * q + 2, set_x)
            if with_count:
                count_rows(day)
            g0.wait()
            pltpu.async_copy(rows_a, acc.at[day], ss_a, add=True)
            if with_count:
                count_rows(dby)
            g1.wait()
            pltpu.async_copy(rows_b, acc.at[dby], ss_b, add=True)

        wait_idx(set_x)
        drain_scatter(ss_a, rows_a)
        drain_scatter(ss_b, rows_b)
        pltpu.sync_copy(table_hbm.at[sax], rows_a)
        pltpu.sync_copy(rows_a, acc.at[dax], add=True)
        if with_count:
            count_rows(dax)

        plsc.subcore_barrier()

        @pl.loop(0, (NCH + NS - 1) // NS)
        def _(j):
            c = j * NS + sid

            @pl.when(c < NCH)
            def _():
                pltpu.sync_copy(acc.at[pl.ds(c * CH, CH)],
                                out_hbm.at[cid, pl.ds(c * CH, CH)])

        if with_count:
            pltpu.sync_copy(cacc.at[pl.ds(sid * CB_PS, CB_PS)],
                            cnt_hbm.at[cid, pl.ds(sid * CB_PS, CB_PS)])

    if with_count:
        return agg
    return lambda *a: agg(*a)[0]


NB = 1000


def _invc(cntp_ref):
    cnt = cntp_ref[0] + cntp_ref[1]
    return 1.0 / jnp.maximum(cnt, 1.0)


_BF = jnp.bfloat16


def _lin1_body(aggp, cntp, x, w1l, w1r, b1, out):
    mean = (aggp[0] + aggp[1]) * _invc(cntp)
    h = (jnp.dot(mean.astype(_BF), w1l[...], preferred_element_type=jnp.float32)
         + jnp.dot(x[...].astype(_BF), w1r[...], preferred_element_type=jnp.float32)
         + b1[...])
    h = jnp.maximum(h, 0.0)
    out[0] = h[:, :D_FEAT]
    out[1] = h[:, D_FEAT:]


def _lin2_body(a2a, a2b, cntp, h1, w2l, w2r, b2, w3l, w3r, b3, p_out, s_out):
    invc = _invc(cntp)
    mean = jnp.concatenate([(a2a[0] + a2a[1]) * invc,
                            (a2b[0] + a2b[1]) * invc], axis=1).astype(_BF)
    hin = jnp.concatenate([h1[0], h1[1]], axis=1).astype(_BF)
    h = (jnp.dot(mean, w2l[...], preferred_element_type=jnp.float32)
         + jnp.dot(hin, w2r[...], preferred_element_type=jnp.float32)
         + b2[...])
    h = jnp.maximum(h, 0.0).astype(_BF)
    p_out[...] = jnp.dot(h, w3l[...], preferred_element_type=jnp.float32)
    s_out[...] = jnp.dot(h, w3r[...], preferred_element_type=jnp.float32) + b3[...]


def _fin_body(a3, cntp, s, out):
    out[...] = (a3[0, :, :NUM_CLASSES] + a3[1, :, :NUM_CLASSES]) * _invc(cntp) + s[...]


def _full(shape):
    return pl.BlockSpec(shape, lambda i: (0,) * len(shape))


def _rows(shape):
    if len(shape) == 3:
        return pl.BlockSpec(shape, lambda i: (0, i, 0))
    return pl.BlockSpec(shape, lambda i: (i, 0))


def kernel(x, edge_index, W1l, W1r, b1, W2l, W2r, b2, W3l, W3r, b3):
    ei = edge_index.astype(jnp.int32)
    src = jnp.pad(ei[0], (0, 2 * K))
    dst = jnp.pad(ei[1], (0, 2 * K))
    b1r, b2r, b3r = b1.reshape(1, -1), b2.reshape(1, -1), b3.reshape(1, -1)
    W3lp = jnp.pad(W3l, ((0, 0), (0, D_FEAT - NUM_CLASSES)))
    W1lb, W1rb, W2lb, W2rb = (w.astype(_BF) for w in (W1l, W1r, W2l, W2r))
    W3lb, W3rb = W3lp.astype(_BF), W3r.astype(_BF)

    agg1p, cbins = _make_agg(D_FEAT, True)(x, src, dst)
    cntp = cbins[:, :N_NODES // 16, :16].reshape(2, N_NODES, 1)

    h1fm = pl.pallas_call(
        _lin1_body,
        grid=(N_NODES // NB,),
        in_specs=[_rows((2, NB, D_FEAT)), _rows((2, NB, 1)),
                  _rows((NB, D_FEAT)), _full((D_FEAT, HIDDEN)),
                  _full((D_FEAT, HIDDEN)), _full((1, HIDDEN))],
        out_specs=_rows((2, NB, D_FEAT)),
        out_shape=jax.ShapeDtypeStruct((2, N_NODES, D_FEAT), jnp.float32),
    )(agg1p, cntp, x, W1lb, W1rb, b1r)

    agg2 = _make_agg(D_FEAT, False)
    a2a = agg2(h1fm[0], src, dst)
    a2b = agg2(h1fm[1], src, dst)

    p, s = pl.pallas_call(
        _lin2_body,
        grid=(N_NODES // NB,),
        in_specs=[_rows((2, NB, D_FEAT)), _rows((2, NB, D_FEAT)),
                  _rows((2, NB, 1)), _rows((2, NB, D_FEAT)),
                  _full((HIDDEN, 2 * HIDDEN)), _full((HIDDEN, 2 * HIDDEN)),
                  _full((1, 2 * HIDDEN)), _full((2 * HIDDEN, D_FEAT)),
                  _full((2 * HIDDEN, NUM_CLASSES)), _full((1, NUM_CLASSES))],
        out_specs=[_rows((NB, D_FEAT)), _rows((NB, NUM_CLASSES))],
        out_shape=[jax.ShapeDtypeStruct((N_NODES, D_FEAT), jnp.float32),
                   jax.ShapeDtypeStruct((N_NODES, NUM_CLASSES), jnp.float32)],
    )(a2a, a2b, cntp, h1fm, W2lb, W2rb, b2r, W3lb, W3rb, b3r)

    a3 = _make_agg(D_FEAT, False)(p, src, dst)

    out = pl.pallas_call(
        _fin_body,
        grid=(N_NODES // NB,),
        in_specs=[_rows((2, NB, D_FEAT)), _rows((2, NB, 1)),
                  _rows((NB, NUM_CLASSES))],
        out_specs=_rows((NB, NUM_CLASSES)),
        out_shape=jax.ShapeDtypeStruct((N_NODES, NUM_CLASSES), jnp.float32),
    )(a3, cntp, s)

    return out


# speedup vs baseline: 11.1600x; 1.0005x over previous
import functools

import jax
import jax.numpy as jnp
from jax import lax
from jax.experimental import pallas as pl
from jax.experimental.pallas import tpu as pltpu
from jax.experimental.pallas import tpu_sc as plsc

N_NODES = 10000
N_EDGES = 320000
D_FEAT = 128
HIDDEN = 256
NUM_CLASSES = 64

NC = 2
NS = 16
LANES = 16
K = 80
CH = 80
NCH = N_NODES // CH
CBINS = 640
CB_PS = CBINS // NS


def _make_agg(W: int, with_count: bool):
    epw = N_EDGES // (NC * NS)
    nchunks = epw // K
    npairs = nchunks // 2
    mesh = plsc.VectorSubcoreMesh(core_axis_name="c", subcore_axis_name="s")

    out_type = [jax.ShapeDtypeStruct((NC, N_NODES, W), jnp.float32)]
    scratch = [
        pltpu.VMEM((K, W), jnp.float32),
        pltpu.VMEM((K, W), jnp.float32),
        pltpu.VMEM((K,), jnp.int32),
        pltpu.VMEM((K,), jnp.int32),
        pltpu.VMEM((K,), jnp.int32),
        pltpu.VMEM((K,), jnp.int32),
        pltpu.VMEM((K,), jnp.int32),
        pltpu.VMEM((K,), jnp.int32),
        pltpu.VMEM((K,), jnp.int32),
        pltpu.VMEM((K,), jnp.int32),
        pltpu.SemaphoreType.DMA,
        pltpu.SemaphoreType.DMA,
        pltpu.SemaphoreType.DMA,
        pltpu.SemaphoreType.DMA,
        pltpu.SemaphoreType.DMA,
        pltpu.SemaphoreType.DMA,
        pltpu.SemaphoreType.DMA,
        pltpu.SemaphoreType.DMA,
        pltpu.VMEM_SHARED((N_NODES, W), jnp.float32),
    ]
    if with_count:
        out_type.append(jax.ShapeDtypeStruct((NC, CBINS, 128), jnp.float32))
        scratch += [
            pltpu.VMEM((K, 128), jnp.float32),
            pltpu.VMEM((K,), jnp.int32),
            pltpu.VMEM_SHARED((CBINS, 128), jnp.float32),
        ]

    @functools.partial(pl.kernel, mesh=mesh, out_type=out_type,
                       scratch_types=scratch)
    def agg(table_hbm, src_hbm, dst_hbm, out_hbm, *rest):
        if with_count:
            cnt_hbm = rest[0]
            (rows_a, rows_b, sax, dax, sbx, dbx, say, day, sby, dby,
             gs_a, gs_b, ss_a, ss_b, is_a, id_a, is_b, id_b,
             acc, crows, divv, cacc) = rest[1:]
        else:
            (rows_a, rows_b, sax, dax, sbx, dbx, say, day, sby, dby,
             gs_a, gs_b, ss_a, ss_b, is_a, id_a, is_b, id_b, acc) = rest
        set_x = (sax, dax, sbx, dbx)
        set_y = (say, day, sby, dby)
        zbuf = rows_a

        cid = lax.axis_index("c")
        sid = lax.axis_index("s")
        wid = sid * NC + cid
        base = wid * epw

        zeros = jnp.zeros((LANES,), jnp.float32)
        iota16 = lax.iota(jnp.int32, LANES)

        @pl.loop(0, K)
        def _(i):
            @pl.loop(0, W, step=LANES)
            def _(j):
                zbuf[i, pl.ds(j, LANES)] = zeros

        if with_count:
            @pl.loop(0, K)
            def _(i):
                @pl.loop(0, 128, step=LANES)
                def _(j):
                    crows[i, pl.ds(j, LANES)] = zeros

            pltpu.sync_copy(crows.at[pl.ds(0, CB_PS)],
                            cacc.at[pl.ds(sid * CB_PS, CB_PS)])

        @pl.loop(0, (NCH + NS - 1) // NS)
        def _(j):
            c = j * NS + sid

            @pl.when(c < NCH)
            def _():
                pltpu.sync_copy(zbuf, acc.at[pl.ds(c * CH, CH)])

        plsc.subcore_barrier()

        isems = (is_a, id_a, is_b, id_b)

        def prefetch(p, bufs):
            off = base + p * 2 * K
            pltpu.async_copy(src_hbm.at[pl.ds(off, K)], bufs[0], is_a)
            pltpu.async_copy(dst_hbm.at[pl.ds(off, K)], bufs[1], id_a)
            pltpu.async_copy(src_hbm.at[pl.ds(off + K, K)], bufs[2], is_b)
            pltpu.async_copy(dst_hbm.at[pl.ds(off + K, K)], bufs[3], id_b)

        def wait_idx(bufs):
            for buf, sem in zip(bufs, isems):
                pltpu.make_async_copy(src_hbm.at[pl.ds(base, K)], buf,
                                      sem).wait()

        def drain_scatter(sem, rows):
            pltpu.make_async_copy(table_hbm.at[pl.ds(0, K)], rows,
                                  sem).wait()

        def count_rows(dv):
            @pl.loop(0, K, step=LANES)
            def _(i):
                dvec = dv[pl.ds(i, LANES)]
                divv[pl.ds(i, LANES)] = dvec >> 4
                dm = dvec & 15
                for l in range(LANES):
                    crows[i + l, pl.ds(0, LANES)] = jnp.where(
                        iota16 == dm[l], 1.0, 0.0)

            pltpu.sync_copy(crows, cacc.at[divv], add=True)

        prefetch(0, set_x)

        @pl.loop(0, npairs // 2)
        def _(q):
            wait_idx(set_x)

            @pl.when(q > 0)
            def _():
                drain_scatter(ss_a, rows_a)

            g0 = pltpu.async_copy(table_hbm.at[sax], rows_a, gs_a)

            @pl.when(q > 0)
            def _():
                drain_scatter(ss_b, rows_b)

            g1 = pltpu.async_copy(table_hbm.at[sbx], rows_b, gs_b)
            prefetch(2 * q + 1, set_y)
            if with_count:
                count_rows(dax)
            g0.wait()
            pltpu.async_copy(rows_a, acc.at[dax], ss_a, add=True)
            if with_count:
                count_rows(dbx)
            g1.wait()
            pltpu.async_copy(rows_b, acc.at[dbx], ss_b, add=True)

            wait_idx(set_y)
            drain_scatter(ss_a, rows_a)
            g0 = pltpu.async_copy(table_hbm.at[say], rows_a, gs_a)
            drain_scatter(ss_b, rows_b)
            g1 = pltpu.async_copy(table_hbm.at[sby], rows_b, gs_b)
            prefetch(2 * q + 2, set_x)
            if with_count:
                count_rows(day)
            g0.wait()
            pltpu.async_copy(rows_a, acc.at[day], ss_a, add=True)
            if with_count:
                count_rows(dby)
            g1.wait()
            pltpu.async_copy(rows_b, acc.at[dby], ss_b, add=True)

        wait_idx(set_x)
        drain_scatter(ss_a, rows_a)
        drain_scatter(ss_b, rows_b)
        pltpu.sync_copy(table_hbm.at[sax], rows_a)
        pltpu.sync_copy(rows_a, acc.at[dax], add=True)
        if with_count:
            count_rows(dax)

        plsc.subcore_barrier()

        @pl.loop(0, (NCH + NS - 1) // NS)
        def _(j):
            c = j * NS + sid

            @pl.when(c < NCH)
            def _():
                pltpu.sync_copy(acc.at[pl.ds(c * CH, CH)],
                                out_hbm.at[cid, pl.ds(c * CH, CH)])

        if with_count:
            pltpu.sync_copy(cacc.at[pl.ds(sid * CB_PS, CB_PS)],
                            cnt_hbm.at[cid, pl.ds(sid * CB_PS, CB_PS)])

    if with_count:
        return agg
    return lambda *a: agg(*a)[0]


NB = 1000


def _invc(cntp_ref):
    cnt = cntp_ref[0] + cntp_ref[1]
    return 1.0 / jnp.maximum(cnt, 1.0)


def _lin1_body(aggp, cntp, x, w1l, w1r, b1, out):
    mean = (aggp[0] + aggp[1]) * _invc(cntp)
    h = (jnp.dot(mean, w1l[...], preferred_element_type=jnp.float32)
         + jnp.dot(x[...], w1r[...], preferred_element_type=jnp.float32)
         + b1[...])
    h = jnp.maximum(h, 0.0)
    out[0] = h[:, :D_FEAT]
    out[1] = h[:, D_FEAT:]


def _lin2_body(a2a, a2b, cntp, h1, w2l, w2r, b2, w3l, w3r, b3, p_out, s_out):
    invc = _invc(cntp)
    mean = jnp.concatenate([(a2a[0] + a2a[1]) * invc,
                            (a2b[0] + a2b[1]) * invc], axis=1)
    hin = jnp.concatenate([h1[0], h1[1]], axis=1)
    h = (jnp.dot(mean, w2l[...], preferred_element_type=jnp.float32)
         + jnp.dot(hin, w2r[...], preferred_element_type=jnp.float32)
         + b2[...])
    h = jnp.maximum(h, 0.0)
    p_out[...] = jnp.dot(h, w3l[...], preferred_element_type=jnp.float32)
    s_out[...] = jnp.dot(h, w3r[...], preferred_element_type=jnp.float32) + b3[...]


def _fin_body(a3, cntp, s, out):
    out[...] = (a3[0, :, :NUM_CLASSES] + a3[1, :, :NUM_CLASSES]) * _invc(cntp) + s[...]


def _full(shape):
    return pl.BlockSpec(shape, lambda i: (0,) * len(shape))


def _rows(shape):
    if len(shape) == 3:
        return pl.BlockSpec(shape, lambda i: (0, i, 0))
    return pl.BlockSpec(shape, lambda i: (i, 0))


def kernel(x, edge_index, W1l, W1r, b1, W2l, W2r, b2, W3l, W3r, b3):
    ei = edge_index.astype(jnp.int32)
    src = jnp.pad(ei[0], (0, 2 * K))
    dst = jnp.pad(ei[1], (0, 2 * K))
    b1r, b2r, b3r = b1.reshape(1, -1), b2.reshape(1, -1), b3.reshape(1, -1)
    W3lp = jnp.pad(W3l, ((0, 0), (0, D_FEAT - NUM_CLASSES)))

    agg1p, cbins = _make_agg(D_FEAT, True)(x, src, dst)
    cntp = cbins[:, :N_NODES // 16, :16].reshape(2, N_NODES, 1)

    h1fm = pl.pallas_call(
        _lin1_body,
        grid=(N_NODES // NB,),
        in_specs=[_rows((2, NB, D_FEAT)), _rows((2, NB, 1)),
                  _rows((NB, D_FEAT)), _full((D_FEAT, HIDDEN)),
                  _full((D_FEAT, HIDDEN)), _full((1, HIDDEN))],
        out_specs=_rows((2, NB, D_FEAT)),
        out_shape=jax.ShapeDtypeStruct((2, N_NODES, D_FEAT), jnp.float32),
    )(agg1p, cntp, x, W1l, W1r, b1r)

    agg2 = _make_agg(D_FEAT, False)
    a2a = agg2(h1fm[0], src, dst)
    a2b = agg2(h1fm[1], src, dst)

    p, s = pl.pallas_call(
        _lin2_body,
        grid=(N_NODES // NB,),
        in_specs=[_rows((2, NB, D_FEAT)), _rows((2, NB, D_FEAT)),
                  _rows((2, NB, 1)), _rows((2, NB, D_FEAT)),
                  _full((HIDDEN, 2 * HIDDEN)), _full((HIDDEN, 2 * HIDDEN)),
                  _full((1, 2 * HIDDEN)), _full((2 * HIDDEN, D_FEAT)),
                  _full((2 * HIDDEN, NUM_CLASSES)), _full((1, NUM_CLASSES))],
        out_specs=[_rows((NB, D_FEAT)), _rows((NB, NUM_CLASSES))],
        out_shape=[jax.ShapeDtypeStruct((N_NODES, D_FEAT), jnp.float32),
                   jax.ShapeDtypeStruct((N_NODES, NUM_CLASSES), jnp.float32)],
    )(a2a, a2b, cntp, h1fm, W2l, W2r, b2r, W3lp, W3r, b3r)

    a3 = _make_agg(D_FEAT, False)(p, src, dst)

    out = pl.pallas_call(
        _fin_body,
        grid=(N_NODES // NB,),
        in_specs=[_rows((2, NB, D_FEAT)), _rows((2, NB, 1)),
                  _rows((NB, NUM_CLASSES))],
        out_specs=_rows((NB, NUM_CLASSES)),
        out_shape=jax.ShapeDtypeStruct((N_NODES, NUM_CLASSES), jnp.float32),
    )(a3, cntp, s)

    return out
